# Initial kernel scaffold; baseline (speedup 1.0000x reference)
import jax
import jax.numpy as jnp
from jax.experimental import pallas as pl


def kernel(bbox_pred, iou_pred, score_pred, targets):
    raise NotImplementedError("write your pallas kernel here")


# speedup vs baseline: 14.8693x; 14.8693x over previous
import functools

import numpy as np
import jax
import jax.numpy as jnp
from jax import lax
from jax.experimental import pallas as pl
from jax.experimental.pallas import tpu as pltpu
from jax.experimental.pallas import tpu_sc as plsc

S = 19
A = 5
C = 20
B = 32
NGT = 16
HW = S * S
NCHUNK = 23
HWP = NCHUNK * 16
_ANCH = np.array(
    [[1.3221, 1.73145], [3.19275, 4.00944], [5.05587, 8.09892],
     [9.47112, 4.84053], [11.2364, 10.0071]], dtype=np.float32) / 17.0
_BASE = np.array([0.5, 0.5, 1.0, 1.0], dtype=np.float32)
_INV_S = float(np.float32(1.0) / np.float32(S))
_AWS = [float(np.float32(_ANCH[a, 0]) * np.float32(_INV_S)) for a in range(A)]
_AHS = [float(np.float32(_ANCH[a, 1]) * np.float32(_INV_S)) for a in range(A)]

_NC = 2
_NS = 16


def _full(v, dtype=jnp.int32):
    return jnp.full((16,), v, dtype=dtype)


_GDN = lax.GatherDimensionNumbers(
    offset_dims=(), collapsed_slice_dims=(0,), start_index_map=(0,))


def _take(vec, idx):
    return lax.gather(vec, idx[:, None], dimension_numbers=_GDN,
                      slice_sizes=(1,),
                      mode=lax.GatherScatterMode.PROMISE_IN_BOUNDS)


def _body(bb_hbm, iou_hbm, tgt_hbm, score_hbm, cm_hbm, out_hbm,
          tv, bb_v, iou_v, i4_v, sp_v, cm_v, rows_v, ov_v,
          sem_a, sem_b, sem_c, sem_sc):
    cidx = lax.axis_index("c")
    sidx = lax.axis_index("s")
    wid = sidx * _NC + cidx

    cp_sc = pltpu.async_copy(score_hbm.at[wid], rows_v, sem_sc)
    cp_bb = pltpu.async_copy(bb_hbm.at[wid], bb_v, sem_a)
    cp_iou = pltpu.async_copy(iou_hbm.at[wid], iou_v, sem_b)
    cp_cm = pltpu.async_copy(cm_hbm, cm_v, sem_c)
    pltpu.sync_copy(tgt_hbm.at[wid], tv)

    iota = lax.iota(jnp.int32, 16)

    cls_i = tv[1].astype(jnp.int32)
    x1 = tv[2]
    y1 = tv[3]
    x2 = tv[4]
    y2 = tv[5]
    gw = x2 - x1
    gh = y2 - y1
    cx = (x1 + x2) / 2.0 * float(S)
    cy = (y1 + y2) / 2.0 * float(S)
    ci = cx.astype(jnp.int32)
    cj = cy.astype(jnp.int32)
    fx = cx - ci.astype(jnp.float32)
    fy = cy - cj.astype(jnp.float32)
    cell = ci * S + cj

    dup = iota < 0
    for k in range(NGT - 1):
        bc = _take(cell, _full(k))
        dup = jnp.logical_or(dup, jnp.logical_and(iota > k, cell == bc))
    keep = jnp.logical_not(dup)
    cs = plsc.cumsum(jnp.where(keep, 1.0, 0.0))
    nk_f = _take(cs, _full(NGT - 1))
    valid = iota.astype(jnp.float32) < nk_f

    aw0 = float(_ANCH[0, 0])
    ah0 = float(_ANCH[0, 1])
    best_r = jnp.minimum(aw0, gw) * jnp.minimum(ah0, gh)
    best_r = best_r / (aw0 * ah0 + gw * gh - best_r)
    aidx = _full(0)
    for a in range(1, A):
        aw = float(_ANCH[a, 0])
        ah = float(_ANCH[a, 1])
        inter = jnp.minimum(aw, gw) * jnp.minimum(ah, gh)
        r = inter / (aw * ah + gw * gh - inter)
        upd = r > best_r
        aidx = jnp.where(upd, a, aidx)
        best_r = jnp.where(upd, r, best_r)

    key = jnp.where(keep, iota, iota + NGT)
    _, perm = plsc.sort_key_val(key, iota)
    cells_s, _ = plsc.sort_key_val(jnp.where(keep, cell, HW), iota)

    a_j = _take(aidx, perm)
    row_l = jnp.where(valid, cells_s * A + a_j, 0)

    area_b = gw * gh
    sx1 = jnp.where(keep, x1, 4.0)
    sy1 = jnp.where(keep, y1, 4.0)
    sx2 = jnp.where(keep, x2, 4.0)
    sy2 = jnp.where(keep, y2, 4.0)
    sab = jnp.where(keep, area_b, 0.0)
    for g in range(NGT):
        gi = _full(g)
        sp_v[0, g] = _take(sx1, gi)
        sp_v[1, g] = _take(sy1, gi)
        sp_v[2, g] = _take(sx2, gi)
        sp_v[3, g] = _take(sy2, gi)
        sp_v[4, g] = _take(sab, gi)

    cp_bb.wait()
    cp_iou.wait()
    cp_cm.wait()

    accb = jnp.zeros((16,), jnp.float32)
    acci = jnp.zeros((16,), jnp.float32)
    for a in range(A):
        aws = _AWS[a]
        ahs = _AHS[a]

        def chunk(t, carry, a=a, aws=aws, ahs=ahs):
            accb, acci = carry
            sl = pl.ds(t * 16, 16)
            bx = bb_v[a, 0, sl]
            by = bb_v[a, 1, sl]
            bw = bb_v[a, 2, sl]
            bh = bb_v[a, 3, sl]
            pcx = (bx + cm_v[0, sl]) * _INV_S
            pcy = (by + cm_v[1, sl]) * _INV_S
            hx = bw * (aws * 0.5)
            hy = bh * (ahs * 0.5)
            px1 = pcx - hx
            py1 = pcy - hy
            px2 = pcx + hx
            py2 = pcy + hy
            area_p = (px2 - px1) * (py2 - py1)
            m = jnp.full((16,), -3.0e38, jnp.float32)
            for g in range(NGT):
                lt1 = jnp.maximum(px1, sp_v[0, g])
                lt2 = jnp.maximum(py1, sp_v[1, g])
                rb1 = jnp.minimum(px2, sp_v[2, g])
                rb2 = jnp.minimum(py2, sp_v[3, g])
                w_ = jnp.maximum(rb1 - lt1, 0.0)
                h_ = jnp.maximum(rb2 - lt2, 0.0)
                inter = w_ * h_
                union = (area_p + sp_v[4, g]) - inter
                m = jnp.maximum(m, inter - 0.6 * union)
            q = iou_v[a, sl]
            q2 = q * q
            v4 = jnp.where(m <= 0.0, q2 * q2, 0.0)
            i4_v[a, sl] = v4
            acci = acci + v4
            d0 = 0.01 * bx - 0.005
            d1 = 0.01 * by - 0.005
            d2 = 0.01 * bw - 0.01
            d3 = 0.01 * bh - 0.01
            accb = accb + (d0 * d0 + d1 * d1) + (d2 * d2 + d3 * d3)
            return accb, acci

        accb, acci = lax.fori_loop(0, NCHUNK, chunk, (accb, acci))

    c_j = cells_s
    bxm = plsc.load_gather(bb_v, [a_j, _full(0), c_j])
    bym = plsc.load_gather(bb_v, [a_j, _full(1), c_j])
    bwm = plsc.load_gather(bb_v, [a_j, _full(2), c_j])
    bhm = plsc.load_gather(bb_v, [a_j, _full(3), c_j])
    cif_m = (c_j // S).astype(jnp.float32)
    cjf_m = (c_j % S).astype(jnp.float32)
    awm = jnp.full((16,), _AWS[0], jnp.float32)
    ahm = jnp.full((16,), _AHS[0], jnp.float32)
    for a in range(1, A):
        awm = jnp.where(a_j == a, _AWS[a], awm)
        ahm = jnp.where(a_j == a, _AHS[a], ahm)
    pcx = (bxm + cif_m) * _INV_S
    pcy = (bym + cjf_m) * _INV_S
    hx = bwm * awm * 0.5
    hy = bhm * ahm * 0.5
    px1 = pcx - hx
    py1 = pcy - hy
    px2 = pcx + hx
    py2 = pcy + hy
    area_p = (px2 - px1) * (py2 - py1)
    gx1 = _take(x1, perm)
    gy1 = _take(y1, perm)
    gx2 = _take(x2, perm)
    gy2 = _take(y2, perm)
    gab = _take(area_b, perm)
    lt1 = jnp.maximum(px1, gx1)
    lt2 = jnp.maximum(py1, gy1)
    rb1 = jnp.minimum(px2, gx2)
    rb2 = jnp.minimum(py2, gy2)
    w_ = jnp.maximum(rb1 - lt1, 0.0)
    h_ = jnp.maximum(rb2 - lt2, 0.0)
    inter = w_ * h_
    tgt_iou = inter / ((area_p + gab) - inter + 1e-12)
    q_m = plsc.load_gather(iou_v, [a_j, c_j])
    i4_m = plsc.load_gather(i4_v, [a_j, c_j])
    im = 5.0 * (1.0 - q_m)
    di = q_m * im - tgt_iou * im
    corr = jnp.where(valid, di * di - i4_m, 0.0)

    tarx = _take(fx, perm)
    tary = _take(fy, perm)
    tarw = _take(gw, perm)
    tarh = _take(gh, perm)
    cb = jnp.zeros((16,), jnp.float32)
    for bv, tar, base in ((bxm, tarx, 0.5), (bym, tary, 0.5),
                          (bwm, tarw, 1.0), (bhm, tarh, 1.0)):
        d1 = bv - tar
        d0 = 0.01 * bv - 0.01 * base
        cb = cb + (d1 * d1 - d0 * d0)
    corr = corr + jnp.where(valid, cb, 0.0)

    cp_sc.wait()
    cls_m = _take(cls_i, perm)
    acc_cls = jnp.zeros((16,), jnp.float32)
    for cc in range(C):
        scol = plsc.load_gather(rows_v, [row_l, _full(cc)])
        d = jnp.where(cls_m == cc, scol - 1.0, scol)
        acc_cls = acc_cls + d * d
    corr = corr + jnp.where(valid, acc_cls, 0.0)

    ov_v[...] = (accb + acci + corr) / nk_f
    pltpu.sync_copy(ov_v, out_hbm.at[wid])


@jax.jit
def _run(bb_t, iou_t, tgt_t, score_flat, cm):
    mesh = plsc.VectorSubcoreMesh(core_axis_name="c", subcore_axis_name="s")
    f = pl.kernel(
        _body,
        out_type=jax.ShapeDtypeStruct((B, 16), jnp.float32),
        mesh=mesh,
        scratch_types=[
            pltpu.VMEM((6, 16), jnp.float32),
            pltpu.VMEM((A, 4, HWP), jnp.float32),
            pltpu.VMEM((A, HWP), jnp.float32),
            pltpu.VMEM((A, HWP), jnp.float32),
            pltpu.VMEM((5, NGT, 16), jnp.float32),
            pltpu.VMEM((2, HWP), jnp.float32),
            pltpu.VMEM((HW * A, C), jnp.float32),
            pltpu.VMEM((16,), jnp.float32),
            pltpu.SemaphoreType.DMA,
            pltpu.SemaphoreType.DMA,
            pltpu.SemaphoreType.DMA,
            pltpu.SemaphoreType.DMA,
        ],
        compiler_params=pltpu.CompilerParams(
            needs_layout_passes=False, use_tc_tiling_on_sc=False),
    )
    return f(bb_t, iou_t, tgt_t, score_flat, cm)


def kernel(bbox_pred, iou_pred, score_pred, targets):
    bb_t = jnp.transpose(bbox_pred.reshape(B, HW, A, 4), (0, 2, 3, 1))
    padc = jnp.broadcast_to(jnp.asarray(_BASE)[None, None, :, None],
                            (B, A, 4, HWP - HW))
    bb_t = jnp.concatenate([bb_t, padc], axis=3)
    iou_t = jnp.transpose(iou_pred.reshape(B, HW, A), (0, 2, 1))
    iou_t = jnp.concatenate(
        [iou_t, jnp.zeros((B, A, HWP - HW), jnp.float32)], axis=2)
    tgt_t = jnp.transpose(targets.reshape(B, NGT, 6), (0, 2, 1))
    score_flat = score_pred.reshape(B, HW * A, C)
    n = np.arange(HWP)
    cm = jnp.asarray(np.stack([
        np.where(n < HW, n // S, 0).astype(np.float32),
        np.where(n < HW, n % S, 0).astype(np.float32)]))
    out = _run(bb_t, iou_t, tgt_t, score_flat, cm)
    return jnp.sum(out)
